# Initial kernel scaffold; baseline (speedup 1.0000x reference)
import jax
import jax.numpy as jnp
from jax.experimental import pallas as pl


def kernel(user_indices, item_indices, util_rows, util_cols, util_vals, user_emb, item_emb, user_avg, movie_avg, global_mean):
    raise NotImplementedError("write your pallas kernel here")


# speedup vs baseline: 70.7567x; 70.7567x over previous
import functools

import jax
import jax.numpy as jnp
from jax import lax
from jax.experimental import pallas as pl
from jax.experimental.pallas import tpu as pltpu
from jax.experimental.pallas import tpu_sc as plsc

LAMBDA1 = 0.1
LAMBDA2 = 0.1
TOP_K = 256

_NC = 2
_NS = 16
_NW = _NC * _NS
_L = 16

_CHUNK = 10240


def _sc_stage(ui, ii, ur, uc, uv, user_emb, item_emb, user_avg, movie_avg, gm16):
    B = ui.shape[0]
    NNZ = ur.shape[0]
    NU, D = user_emb.shape
    NI = item_emb.shape[0]
    QB = B // _NW
    assert B % (_L * _NW) == 0 and NI % _L == 0 and NU % _L == 0
    assert NNZ % _L == 0

    ch_main = (NNZ // _NW) & ~(_L - 1)
    tail = NNZ - _NW * ch_main
    assert tail % _L == 0 and tail <= _CHUNK
    chunks = [_CHUNK] * (ch_main // _CHUNK)
    if ch_main % _CHUNK:
        chunks.append(ch_main % _CHUNK)

    mesh = plsc.VectorSubcoreMesh(core_axis_name="c", subcore_axis_name="s",
                                  num_cores=_NC, num_subcores=_NS)

    @functools.partial(
        pl.kernel,
        mesh=mesh,
        compiler_params=pltpu.CompilerParams(needs_layout_passes=False),
        out_type=[
            jax.ShapeDtypeStruct((_NW, NI), jnp.float32),
            jax.ShapeDtypeStruct((B,), jnp.float32),
            jax.ShapeDtypeStruct((B,), jnp.float32),
            jax.ShapeDtypeStruct((B, D), jnp.float32),
            jax.ShapeDtypeStruct((B, D), jnp.float32),
        ],
        scratch_types=[
            pltpu.VMEM((NU,), jnp.float32),
            pltpu.VMEM((NI,), jnp.float32),
            pltpu.VMEM((NI,), jnp.float32),
            pltpu.VMEM((_L,), jnp.float32),
            pltpu.VMEM((QB,), jnp.int32),
            pltpu.VMEM((QB,), jnp.int32),
            pltpu.VMEM((QB, D), jnp.float32),
            pltpu.VMEM((QB, D), jnp.float32),
            pltpu.VMEM((QB,), jnp.float32),
            pltpu.VMEM((QB,), jnp.float32),
            pltpu.VMEM((_CHUNK,), jnp.int32),
            pltpu.VMEM((_CHUNK,), jnp.int32),
            pltpu.VMEM((_CHUNK,), jnp.float32),
            pltpu.SemaphoreType.DMA,
        ],
    )
    def sc_kernel(ui_h, ii_h, ur_h, uc_h, uv_h, ue_h, ie_h, ua_h, ma_h, gm_h,
                  colp_o, uq_o, mq_o, ulat_o, ilat_o,
                  uavg_t, mavg_t, colp_v, gm_v, uidx_v, iidx_v, ulat_v,
                  ilat_v, qa_v, qb_v, rows_v, cols_v, vals_v, sem):
        wid = lax.axis_index("s") * _NC + lax.axis_index("c")

        pltpu.sync_copy(ua_h, uavg_t)
        pltpu.sync_copy(ma_h, mavg_t)
        pltpu.sync_copy(gm_h, gm_v)

        qbase = wid * QB
        pltpu.sync_copy(ui_h.at[pl.ds(qbase, QB)], uidx_v)
        pltpu.sync_copy(ii_h.at[pl.ds(qbase, QB)], iidx_v)
        pltpu.async_copy(ue_h.at[uidx_v], ulat_v, sem).wait()
        pltpu.async_copy(ie_h.at[iidx_v], ilat_v, sem).wait()
        pltpu.sync_copy(ulat_v, ulat_o.at[pl.ds(qbase, QB)])
        pltpu.sync_copy(ilat_v, ilat_o.at[pl.ds(qbase, QB)])
        for j in range(QB // _L):
            sl = pl.ds(j * _L, _L)
            qa_v[sl] = plsc.load_gather(uavg_t, [uidx_v[sl]])
            qb_v[sl] = plsc.load_gather(mavg_t, [iidx_v[sl]])
        pltpu.sync_copy(qa_v, uq_o.at[pl.ds(qbase, QB)])
        pltpu.sync_copy(qb_v, mq_o.at[pl.ds(qbase, QB)])

        def zero_body(j, carry):
            colp_v[pl.ds(j * _L, _L)] = jnp.zeros((_L,), jnp.float32)
            return carry
        lax.fori_loop(jnp.int32(0), jnp.int32(NI // _L), zero_body,
                      jnp.int32(0))

        gmv = gm_v[...]

        def accum(nvec):
            def body(j, carry):
                sl = pl.ds(j * _L, _L)
                r16 = rows_v[sl]
                c16 = cols_v[sl]
                v16 = vals_v[sl]
                ua16 = plsc.load_gather(uavg_t, [r16])
                ma16 = plsc.load_gather(mavg_t, [c16])
                plsc.addupdate_scatter(colp_v, [c16], v16 - ua16 - ma16 - gmv)
                return carry
            lax.fori_loop(jnp.int32(0), jnp.int32(nvec), body, jnp.int32(0))

        off = wid * ch_main
        for n in chunks:
            pltpu.sync_copy(ur_h.at[pl.ds(off, n)], rows_v.at[pl.ds(0, n)])
            pltpu.sync_copy(uc_h.at[pl.ds(off, n)], cols_v.at[pl.ds(0, n)])
            pltpu.sync_copy(uv_h.at[pl.ds(off, n)], vals_v.at[pl.ds(0, n)])
            accum(n // _L)
            off = off + n

        if tail > 0:
            @pl.when(wid == 0)
            def _():
                toff = _NW * ch_main
                pltpu.sync_copy(ur_h.at[pl.ds(toff, tail)],
                                rows_v.at[pl.ds(0, tail)])
                pltpu.sync_copy(uc_h.at[pl.ds(toff, tail)],
                                cols_v.at[pl.ds(0, tail)])
                pltpu.sync_copy(uv_h.at[pl.ds(toff, tail)],
                                vals_v.at[pl.ds(0, tail)])
                accum(tail // _L)

        pltpu.sync_copy(colp_v, colp_o.at[wid])

    return sc_kernel(ui, ii, ur, uc, uv, user_emb, item_emb, user_avg,
                     movie_avg, gm16)


def _sortable_keys(x):
    b = lax.bitcast_convert_type(x, jnp.int32)
    return jnp.where(b < 0, b ^ jnp.int32(0x7FFFFFFF), b)


def _tc_body(ilat_ref, iemb_ref, colp_ref, ulat_ref, uq_ref, mq_ref, gm_ref,
             pred_ref, reg_ref, *, k):
    step = pl.program_id(0)
    cs = jnp.sum(colp_ref[...], axis=0, keepdims=True)
    il = ilat_ref[...]
    sim = lax.dot_general(il, iemb_ref[...], (((1,), (1,)), ((), ())),
                          precision=lax.Precision.HIGHEST,
                          preferred_element_type=jnp.float32)
    keys = _sortable_keys(sim)
    rb = sim.shape[0]
    kk = jnp.int32(k)

    def sel_body(_, carry):
        t, bv = carry
        cand = t + bv
        cnt = jnp.sum((keys >= cand).astype(jnp.int32), axis=1,
                      keepdims=True, dtype=jnp.int32)
        return jnp.where(cnt >= kk, cand, t), lax.shift_right_logical(bv, jnp.int32(1))

    t0 = jnp.full((rb, 1), jnp.int32(-2147483648))
    thr, _ = lax.fori_loop(jnp.int32(0), jnp.int32(32), sel_body,
                           (t0, jnp.full((rb, 1), jnp.int32(-2147483648))))

    maskge = keys >= thr
    cnt_ge = jnp.sum(maskge.astype(jnp.int32), axis=1, keepdims=True,
                     dtype=jnp.int32)
    wcs = sim * cs
    contrib = jnp.sum(jnp.where(maskge, wcs, 0.0), axis=1)

    ul = ulat_ref[...]
    svd = jnp.sum(il * ul, axis=1)
    base = uq_ref[...] + mq_ref[...] - gm_ref[0, 0]
    pred_ref[...] = jnp.maximum(base + svd + contrib, 0.0)

    @pl.when(step == 0)
    def _():
        reg_ref[...] = jnp.zeros_like(reg_ref)
    reg_ref[...] = reg_ref[...] + (LAMBDA1 * jnp.sum(ul * ul) +
                                   LAMBDA2 * jnp.sum(il * il))

    @pl.when(jnp.max(cnt_ge) > kk)
    def _():
        ni = sim.shape[1]
        gt = keys > thr
        cnt_gt = jnp.sum(gt.astype(jnp.int32), axis=1, keepdims=True,
                         dtype=jnp.int32)
        needed = kk - cnt_gt
        tie = maskge & jnp.logical_not(gt)
        iota = lax.broadcasted_iota(jnp.int32, (1, ni), 1)
        nbits = max(1, (ni - 1).bit_length())

        def idx_body(_, carry):
            p, bv = carry
            cand = p + bv
            h = jnp.sum((tie & (iota < cand)).astype(jnp.int32), axis=1,
                        keepdims=True, dtype=jnp.int32)
            return jnp.where(h < needed, cand, p), lax.shift_right_logical(bv, jnp.int32(1))

        p0 = jnp.zeros((rb, 1), jnp.int32)
        bv0 = jnp.full((rb, 1), jnp.int32(1 << (nbits - 1)))
        pmax, _ = lax.fori_loop(jnp.int32(0), jnp.int32(nbits), idx_body,
                                (p0, bv0))
        sel = gt | (tie & (iota <= pmax))
        contrib2 = jnp.sum(jnp.where(sel, wcs, 0.0), axis=1)
        pred_ref[...] = jnp.maximum(base + svd + contrib2, 0.0)


def kernel(user_indices, item_indices, util_rows, util_cols, util_vals,
           user_emb, item_emb, user_avg, movie_avg, global_mean):
    B = user_indices.shape[0]
    NI, D = item_emb.shape
    D2 = ((D + 127) // 128) * 128
    uep = jnp.pad(user_emb.astype(jnp.float32), ((0, 0), (0, D2 - D)))
    iep = jnp.pad(item_emb.astype(jnp.float32), ((0, 0), (0, D2 - D)))
    ui = user_indices.astype(jnp.int32)
    ii = item_indices.astype(jnp.int32)
    ur = util_rows.astype(jnp.int32)
    uc = util_cols.astype(jnp.int32)
    uv = util_vals.astype(jnp.float32)
    gm = global_mean.astype(jnp.float32)
    gm16 = jnp.broadcast_to(gm.reshape(1), (_L,))

    colp, uq, mq, ulat, ilat = _sc_stage(
        ui, ii, ur, uc, uv, uep, iep, user_avg.astype(jnp.float32),
        movie_avg.astype(jnp.float32), gm16)

    RB = 256
    assert B % RB == 0
    z = lambda i: jnp.int32(0)
    preds, reg = pl.pallas_call(
        functools.partial(_tc_body, k=TOP_K),
        grid=(B // RB,),
        in_specs=[
            pl.BlockSpec((RB, D2), lambda i: (i, z(i))),
            pl.BlockSpec((NI, D2), lambda i: (z(i), z(i))),
            pl.BlockSpec((_NW, NI), lambda i: (z(i), z(i))),
            pl.BlockSpec((RB, D2), lambda i: (i, z(i))),
            pl.BlockSpec((RB,), lambda i: (i,)),
            pl.BlockSpec((RB,), lambda i: (i,)),
            pl.BlockSpec((1, 1), lambda i: (z(i), z(i))),
        ],
        out_specs=[
            pl.BlockSpec((RB,), lambda i: (i,)),
            pl.BlockSpec((1, 1), lambda i: (z(i), z(i))),
        ],
        out_shape=[
            jax.ShapeDtypeStruct((B,), jnp.float32),
            jax.ShapeDtypeStruct((1, 1), jnp.float32),
        ],
    )(ilat, iep, colp, ulat, uq, mq, gm.reshape(1, 1))

    return preds, reg.reshape(())


# speedup vs baseline: 86.0921x; 1.2167x over previous
import functools

import jax
import jax.numpy as jnp
from jax import lax
from jax.experimental import pallas as pl
from jax.experimental.pallas import tpu as pltpu
from jax.experimental.pallas import tpu_sc as plsc

LAMBDA1 = 0.1
LAMBDA2 = 0.1
TOP_K = 256

_NC = 2
_NS = 16
_NW = _NC * _NS
_L = 16

_CHUNK = 10240


def _sc_stage(ui, ii, ur, uc, uv, user_emb, item_emb, user_avg, movie_avg, gm16):
    B = ui.shape[0]
    NNZ = ur.shape[0]
    NU, D = user_emb.shape
    NI = item_emb.shape[0]
    QB = B // _NW
    assert B % (_L * _NW) == 0 and NI % _L == 0 and NU % _L == 0
    assert NNZ % _L == 0

    ch_main = (NNZ // _NW) & ~(_L - 1)
    tail = NNZ - _NW * ch_main
    assert tail % _L == 0 and tail <= _CHUNK
    chunks = [_CHUNK] * (ch_main // _CHUNK)
    if ch_main % _CHUNK:
        chunks.append(ch_main % _CHUNK)

    mesh = plsc.VectorSubcoreMesh(core_axis_name="c", subcore_axis_name="s",
                                  num_cores=_NC, num_subcores=_NS)

    @functools.partial(
        pl.kernel,
        mesh=mesh,
        compiler_params=pltpu.CompilerParams(needs_layout_passes=False),
        out_type=[
            jax.ShapeDtypeStruct((_NW, NI), jnp.float32),
            jax.ShapeDtypeStruct((B,), jnp.float32),
            jax.ShapeDtypeStruct((B,), jnp.float32),
            jax.ShapeDtypeStruct((B, D), jnp.float32),
            jax.ShapeDtypeStruct((B, D), jnp.float32),
        ],
        scratch_types=[
            pltpu.VMEM((NU,), jnp.float32),
            pltpu.VMEM((NI,), jnp.float32),
            pltpu.VMEM((NI,), jnp.float32),
            pltpu.VMEM((_L,), jnp.float32),
            pltpu.VMEM((QB,), jnp.int32),
            pltpu.VMEM((QB,), jnp.int32),
            pltpu.VMEM((QB, D), jnp.float32),
            pltpu.VMEM((QB, D), jnp.float32),
            pltpu.VMEM((QB,), jnp.float32),
            pltpu.VMEM((QB,), jnp.float32),
            pltpu.VMEM((_CHUNK,), jnp.int32),
            pltpu.VMEM((_CHUNK,), jnp.int32),
            pltpu.VMEM((_CHUNK,), jnp.float32),
            pltpu.SemaphoreType.DMA,
        ],
    )
    def sc_kernel(ui_h, ii_h, ur_h, uc_h, uv_h, ue_h, ie_h, ua_h, ma_h, gm_h,
                  colp_o, uq_o, mq_o, ulat_o, ilat_o,
                  uavg_t, mavg_t, colp_v, gm_v, uidx_v, iidx_v, ulat_v,
                  ilat_v, qa_v, qb_v, rows_v, cols_v, vals_v, sem):
        wid = lax.axis_index("s") * _NC + lax.axis_index("c")

        pltpu.sync_copy(ua_h, uavg_t)
        pltpu.sync_copy(ma_h, mavg_t)
        pltpu.sync_copy(gm_h, gm_v)

        qbase = wid * QB
        pltpu.sync_copy(ui_h.at[pl.ds(qbase, QB)], uidx_v)
        pltpu.sync_copy(ii_h.at[pl.ds(qbase, QB)], iidx_v)
        pltpu.async_copy(ue_h.at[uidx_v], ulat_v, sem).wait()
        pltpu.async_copy(ie_h.at[iidx_v], ilat_v, sem).wait()
        pltpu.sync_copy(ulat_v, ulat_o.at[pl.ds(qbase, QB)])
        pltpu.sync_copy(ilat_v, ilat_o.at[pl.ds(qbase, QB)])
        for j in range(QB // _L):
            sl = pl.ds(j * _L, _L)
            qa_v[sl] = plsc.load_gather(uavg_t, [uidx_v[sl]])
            qb_v[sl] = plsc.load_gather(mavg_t, [iidx_v[sl]])
        pltpu.sync_copy(qa_v, uq_o.at[pl.ds(qbase, QB)])
        pltpu.sync_copy(qb_v, mq_o.at[pl.ds(qbase, QB)])

        def zero_body(j, carry):
            colp_v[pl.ds(j * _L, _L)] = jnp.zeros((_L,), jnp.float32)
            return carry
        lax.fori_loop(jnp.int32(0), jnp.int32(NI // _L), zero_body,
                      jnp.int32(0))

        gmv = gm_v[...]

        def accum(nvec):
            def body(j, carry):
                sl = pl.ds(j * _L, _L)
                r16 = rows_v[sl]
                c16 = cols_v[sl]
                v16 = vals_v[sl]
                ua16 = plsc.load_gather(uavg_t, [r16])
                ma16 = plsc.load_gather(mavg_t, [c16])
                plsc.addupdate_scatter(colp_v, [c16], v16 - ua16 - ma16 - gmv)
                return carry
            lax.fori_loop(jnp.int32(0), jnp.int32(nvec), body, jnp.int32(0))

        off = wid * ch_main
        for n in chunks:
            pltpu.sync_copy(ur_h.at[pl.ds(off, n)], rows_v.at[pl.ds(0, n)])
            pltpu.sync_copy(uc_h.at[pl.ds(off, n)], cols_v.at[pl.ds(0, n)])
            pltpu.sync_copy(uv_h.at[pl.ds(off, n)], vals_v.at[pl.ds(0, n)])
            accum(n // _L)
            off = off + n

        if tail > 0:
            @pl.when(wid == 0)
            def _():
                toff = _NW * ch_main
                pltpu.sync_copy(ur_h.at[pl.ds(toff, tail)],
                                rows_v.at[pl.ds(0, tail)])
                pltpu.sync_copy(uc_h.at[pl.ds(toff, tail)],
                                cols_v.at[pl.ds(0, tail)])
                pltpu.sync_copy(uv_h.at[pl.ds(toff, tail)],
                                vals_v.at[pl.ds(0, tail)])
                accum(tail // _L)

        pltpu.sync_copy(colp_v, colp_o.at[wid])

    return sc_kernel(ui, ii, ur, uc, uv, user_emb, item_emb, user_avg,
                     movie_avg, gm16)


def _sortable_keys(x):
    b = lax.bitcast_convert_type(x, jnp.int32)
    return jnp.where(b < 0, b ^ jnp.int32(0x7FFFFFFF), b)


def _tc_body(ilat_ref, iemb_ref, colp_ref, ulat_ref, uq_ref, mq_ref, gm_ref,
             pred_ref, reg_ref, *, k):
    step = pl.program_id(0)
    cs = jnp.sum(colp_ref[...], axis=0, keepdims=True)
    il = ilat_ref[...]
    sim = lax.dot_general(il, iemb_ref[...], (((1,), (1,)), ((), ())),
                          precision=lax.Precision.HIGHEST,
                          preferred_element_type=jnp.float32)
    keys = _sortable_keys(sim)
    rb = sim.shape[0]
    kk = jnp.int32(k)

    def packed_count(ones):
        w = ones.shape[1]
        while w > 128:
            ones = ones[:, : w // 2] + ones[:, w // 2:]
            w //= 2
        return jnp.sum(ones.astype(jnp.int32), axis=1, keepdims=True,
                       dtype=jnp.int32)

    i16_1 = jnp.int16(1)
    i16_0 = jnp.int16(0)
    hk = lax.shift_right_arithmetic(keys, jnp.int32(16)).astype(jnp.int16)

    def hi_body(_, carry):
        t, bv = carry
        cand = t + bv
        cnt = packed_count(jnp.where(hk >= cand.astype(jnp.int16),
                                     i16_1, i16_0))
        return jnp.where(cnt >= kk, cand, t), \
            lax.shift_right_arithmetic(bv, jnp.int32(1))

    t0 = jnp.full((rb, 1), jnp.int32(-32768))
    bv0 = jnp.full((rb, 1), jnp.int32(32768))
    hstar, _ = lax.fori_loop(jnp.int32(0), jnp.int32(16), hi_body, (t0, bv0))

    lu = ((keys & jnp.int32(0xFFFF)) ^ jnp.int32(0x8000)).astype(jnp.int16)
    h16 = hstar.astype(jnp.int16)
    emask = hk == h16
    cnt_ge_h = packed_count(jnp.where(hk >= h16, i16_1, i16_0))
    cnt_gt_h = cnt_ge_h - packed_count(jnp.where(emask, i16_1, i16_0))
    lo_m = jnp.where(emask, lu, jnp.int16(-32768))

    def lo_body(_, carry):
        t, bv = carry
        cand = t + bv
        cnt = cnt_gt_h + packed_count(
            jnp.where(lo_m >= cand.astype(jnp.int16), i16_1, i16_0))
        return jnp.where(cnt >= kk, cand, t), \
            lax.shift_right_arithmetic(bv, jnp.int32(1))

    lstar, _ = lax.fori_loop(jnp.int32(0), jnp.int32(16), lo_body, (t0, bv0))

    thr = lax.shift_left(hstar, jnp.int32(16)) | (
        (lstar ^ jnp.int32(0x8000)) & jnp.int32(0xFFFF))

    maskge = keys >= thr
    cnt_ge = jnp.sum(maskge.astype(jnp.int32), axis=1, keepdims=True,
                     dtype=jnp.int32)
    wcs = sim * cs
    contrib = jnp.sum(jnp.where(maskge, wcs, 0.0), axis=1)

    ul = ulat_ref[...]
    svd = jnp.sum(il * ul, axis=1)
    base = uq_ref[...] + mq_ref[...] - gm_ref[0, 0]
    pred_ref[...] = jnp.maximum(base + svd + contrib, 0.0)

    @pl.when(step == 0)
    def _():
        reg_ref[...] = jnp.zeros_like(reg_ref)
    reg_ref[...] = reg_ref[...] + (LAMBDA1 * jnp.sum(ul * ul) +
                                   LAMBDA2 * jnp.sum(il * il))

    @pl.when(jnp.max(cnt_ge) > kk)
    def _():
        ni = sim.shape[1]
        gt = keys > thr
        cnt_gt = jnp.sum(gt.astype(jnp.int32), axis=1, keepdims=True,
                         dtype=jnp.int32)
        needed = kk - cnt_gt
        tie = maskge & jnp.logical_not(gt)
        iota = lax.broadcasted_iota(jnp.int32, (1, ni), 1)
        nbits = max(1, (ni - 1).bit_length())

        def idx_body(_, carry):
            p, bv = carry
            cand = p + bv
            h = jnp.sum((tie & (iota < cand)).astype(jnp.int32), axis=1,
                        keepdims=True, dtype=jnp.int32)
            return jnp.where(h < needed, cand, p), lax.shift_right_logical(bv, jnp.int32(1))

        p0 = jnp.zeros((rb, 1), jnp.int32)
        bv0 = jnp.full((rb, 1), jnp.int32(1 << (nbits - 1)))
        pmax, _ = lax.fori_loop(jnp.int32(0), jnp.int32(nbits), idx_body,
                                (p0, bv0))
        sel = gt | (tie & (iota <= pmax))
        contrib2 = jnp.sum(jnp.where(sel, wcs, 0.0), axis=1)
        pred_ref[...] = jnp.maximum(base + svd + contrib2, 0.0)


def kernel(user_indices, item_indices, util_rows, util_cols, util_vals,
           user_emb, item_emb, user_avg, movie_avg, global_mean):
    B = user_indices.shape[0]
    NI, D = item_emb.shape
    D2 = ((D + 127) // 128) * 128
    uep = jnp.pad(user_emb.astype(jnp.float32), ((0, 0), (0, D2 - D)))
    iep = jnp.pad(item_emb.astype(jnp.float32), ((0, 0), (0, D2 - D)))
    ui = user_indices.astype(jnp.int32)
    ii = item_indices.astype(jnp.int32)
    ur = util_rows.astype(jnp.int32)
    uc = util_cols.astype(jnp.int32)
    uv = util_vals.astype(jnp.float32)
    gm = global_mean.astype(jnp.float32)
    gm16 = jnp.broadcast_to(gm.reshape(1), (_L,))

    colp, uq, mq, ulat, ilat = _sc_stage(
        ui, ii, ur, uc, uv, uep, iep, user_avg.astype(jnp.float32),
        movie_avg.astype(jnp.float32), gm16)

    RB = 256
    assert B % RB == 0
    z = lambda i: jnp.int32(0)
    preds, reg = pl.pallas_call(
        functools.partial(_tc_body, k=TOP_K),
        grid=(B // RB,),
        in_specs=[
            pl.BlockSpec((RB, D2), lambda i: (i, z(i))),
            pl.BlockSpec((NI, D2), lambda i: (z(i), z(i))),
            pl.BlockSpec((_NW, NI), lambda i: (z(i), z(i))),
            pl.BlockSpec((RB, D2), lambda i: (i, z(i))),
            pl.BlockSpec((RB,), lambda i: (i,)),
            pl.BlockSpec((RB,), lambda i: (i,)),
            pl.BlockSpec((1, 1), lambda i: (z(i), z(i))),
        ],
        out_specs=[
            pl.BlockSpec((RB,), lambda i: (i,)),
            pl.BlockSpec((1, 1), lambda i: (z(i), z(i))),
        ],
        out_shape=[
            jax.ShapeDtypeStruct((B,), jnp.float32),
            jax.ShapeDtypeStruct((1, 1), jnp.float32),
        ],
    )(ilat, iep, colp, ulat, uq, mq, gm.reshape(1, 1))

    return preds, reg.reshape(())


# speedup vs baseline: 99.2845x; 1.1532x over previous
import functools

import jax
import jax.numpy as jnp
from jax import lax
from jax.experimental import pallas as pl
from jax.experimental.pallas import tpu as pltpu
from jax.experimental.pallas import tpu_sc as plsc

LAMBDA1 = 0.1
LAMBDA2 = 0.1
TOP_K = 256

_NC = 2
_NS = 16
_NW = _NC * _NS
_L = 16

_CHUNK = 10240


def _sc_stage(ui, ii, ur, uc, uv, user_emb, item_emb, user_avg, movie_avg, gm16):
    B = ui.shape[0]
    NNZ = ur.shape[0]
    NU, D = user_emb.shape
    NI = item_emb.shape[0]
    QB = B // _NW
    assert B % (_L * _NW) == 0 and NI % _L == 0 and NU % _L == 0
    assert NNZ % _L == 0

    ch_main = (NNZ // _NW) & ~(_L - 1)
    tail = NNZ - _NW * ch_main
    assert tail % _L == 0 and tail <= _CHUNK
    chunks = [_CHUNK] * (ch_main // _CHUNK)
    if ch_main % _CHUNK:
        chunks.append(ch_main % _CHUNK)

    mesh = plsc.VectorSubcoreMesh(core_axis_name="c", subcore_axis_name="s",
                                  num_cores=_NC, num_subcores=_NS)

    @functools.partial(
        pl.kernel,
        mesh=mesh,
        compiler_params=pltpu.CompilerParams(needs_layout_passes=False),
        out_type=[
            jax.ShapeDtypeStruct((_NW, NI), jnp.float32),
            jax.ShapeDtypeStruct((B,), jnp.float32),
            jax.ShapeDtypeStruct((B,), jnp.float32),
            jax.ShapeDtypeStruct((B, D), jnp.float32),
            jax.ShapeDtypeStruct((B, D), jnp.float32),
        ],
        scratch_types=[
            pltpu.VMEM((NU,), jnp.float32),
            pltpu.VMEM((NI,), jnp.float32),
            pltpu.VMEM((NI,), jnp.float32),
            pltpu.VMEM((_L,), jnp.float32),
            pltpu.VMEM((QB,), jnp.int32),
            pltpu.VMEM((QB,), jnp.int32),
            pltpu.VMEM((QB, D), jnp.float32),
            pltpu.VMEM((QB, D), jnp.float32),
            pltpu.VMEM((QB,), jnp.float32),
            pltpu.VMEM((QB,), jnp.float32),
            pltpu.VMEM((_CHUNK,), jnp.int32),
            pltpu.VMEM((_CHUNK,), jnp.int32),
            pltpu.VMEM((_CHUNK,), jnp.float32),
            pltpu.SemaphoreType.DMA,
        ],
    )
    def sc_kernel(ui_h, ii_h, ur_h, uc_h, uv_h, ue_h, ie_h, ua_h, ma_h, gm_h,
                  colp_o, uq_o, mq_o, ulat_o, ilat_o,
                  uavg_t, mavg_t, colp_v, gm_v, uidx_v, iidx_v, ulat_v,
                  ilat_v, qa_v, qb_v, rows_v, cols_v, vals_v, sem):
        wid = lax.axis_index("s") * _NC + lax.axis_index("c")

        pltpu.sync_copy(ua_h, uavg_t)
        pltpu.sync_copy(ma_h, mavg_t)
        pltpu.sync_copy(gm_h, gm_v)

        qbase = wid * QB
        pltpu.sync_copy(ui_h.at[pl.ds(qbase, QB)], uidx_v)
        pltpu.sync_copy(ii_h.at[pl.ds(qbase, QB)], iidx_v)
        pltpu.async_copy(ue_h.at[uidx_v], ulat_v, sem).wait()
        pltpu.async_copy(ie_h.at[iidx_v], ilat_v, sem).wait()
        pltpu.sync_copy(ulat_v, ulat_o.at[pl.ds(qbase, QB)])
        pltpu.sync_copy(ilat_v, ilat_o.at[pl.ds(qbase, QB)])
        for j in range(QB // _L):
            sl = pl.ds(j * _L, _L)
            qa_v[sl] = plsc.load_gather(uavg_t, [uidx_v[sl]])
            qb_v[sl] = plsc.load_gather(mavg_t, [iidx_v[sl]])
        pltpu.sync_copy(qa_v, uq_o.at[pl.ds(qbase, QB)])
        pltpu.sync_copy(qb_v, mq_o.at[pl.ds(qbase, QB)])

        def zero_body(j, carry):
            colp_v[pl.ds(j * _L, _L)] = jnp.zeros((_L,), jnp.float32)
            return carry
        lax.fori_loop(jnp.int32(0), jnp.int32(NI // _L), zero_body,
                      jnp.int32(0))

        gmv = gm_v[...]

        def accum(nvec):
            unroll = 4

            def group(j):
                sl = pl.ds(j * _L, _L)
                r16 = rows_v[sl]
                c16 = cols_v[sl]
                v16 = vals_v[sl]
                ua16 = plsc.load_gather(uavg_t, [r16])
                ma16 = plsc.load_gather(mavg_t, [c16])
                plsc.addupdate_scatter(colp_v, [c16], v16 - ua16 - ma16 - gmv)

            def body(j, carry):
                for u in range(unroll):
                    group(j * unroll + jnp.int32(u))
                return carry

            def body1(j, carry):
                group(j)
                return carry

            lax.fori_loop(jnp.int32(0), jnp.int32(nvec // unroll), body,
                          jnp.int32(0))
            if nvec % unroll:
                lax.fori_loop(jnp.int32(nvec - nvec % unroll),
                              jnp.int32(nvec), body1, jnp.int32(0))

        off = wid * ch_main
        for n in chunks:
            pltpu.sync_copy(ur_h.at[pl.ds(off, n)], rows_v.at[pl.ds(0, n)])
            pltpu.sync_copy(uc_h.at[pl.ds(off, n)], cols_v.at[pl.ds(0, n)])
            pltpu.sync_copy(uv_h.at[pl.ds(off, n)], vals_v.at[pl.ds(0, n)])
            accum(n // _L)
            off = off + n

        if tail > 0:
            @pl.when(wid == 0)
            def _():
                toff = _NW * ch_main
                pltpu.sync_copy(ur_h.at[pl.ds(toff, tail)],
                                rows_v.at[pl.ds(0, tail)])
                pltpu.sync_copy(uc_h.at[pl.ds(toff, tail)],
                                cols_v.at[pl.ds(0, tail)])
                pltpu.sync_copy(uv_h.at[pl.ds(toff, tail)],
                                vals_v.at[pl.ds(0, tail)])
                accum(tail // _L)

        pltpu.sync_copy(colp_v, colp_o.at[wid])

    return sc_kernel(ui, ii, ur, uc, uv, user_emb, item_emb, user_avg,
                     movie_avg, gm16)


def _sortable_keys(x):
    b = lax.bitcast_convert_type(x, jnp.int32)
    return jnp.where(b < 0, b ^ jnp.int32(0x7FFFFFFF), b)


def _tc_body(ilat_ref, iemb_ref, colp_ref, ulat_ref, uq_ref, mq_ref, gm_ref,
             pred_ref, reg_ref, *, k):
    step = pl.program_id(0)
    cs = jnp.sum(colp_ref[...], axis=0, keepdims=True)
    il = ilat_ref[...]
    sim = lax.dot_general(il, iemb_ref[...], (((1,), (1,)), ((), ())),
                          preferred_element_type=jnp.float32)
    keys = _sortable_keys(sim)
    rb = sim.shape[0]
    kk = jnp.int32(k)

    def packed_count(ones):
        w = ones.shape[1]
        while w > 128:
            ones = ones[:, : w // 2] + ones[:, w // 2:]
            w //= 2
        return jnp.sum(ones.astype(jnp.int32), axis=1, keepdims=True,
                       dtype=jnp.int32)

    i16_1 = jnp.int16(1)
    i16_0 = jnp.int16(0)
    hk = lax.shift_right_arithmetic(keys, jnp.int32(16)).astype(jnp.int16)

    def hi_body(_, carry):
        t, bv = carry
        cand = t + bv
        cnt = packed_count(jnp.where(hk >= cand.astype(jnp.int16),
                                     i16_1, i16_0))
        return jnp.where(cnt >= kk, cand, t), \
            lax.shift_right_arithmetic(bv, jnp.int32(1))

    t0 = jnp.full((rb, 1), jnp.int32(-32768))
    bv0 = jnp.full((rb, 1), jnp.int32(32768))
    hstar, _ = lax.fori_loop(jnp.int32(0), jnp.int32(16), hi_body, (t0, bv0))

    lu = ((keys & jnp.int32(0xFFFF)) ^ jnp.int32(0x8000)).astype(jnp.int16)
    h16 = hstar.astype(jnp.int16)
    emask = hk == h16
    cnt_ge_h = packed_count(jnp.where(hk >= h16, i16_1, i16_0))
    cnt_gt_h = cnt_ge_h - packed_count(jnp.where(emask, i16_1, i16_0))
    lo_m = jnp.where(emask, lu, jnp.int16(-32768))

    def lo_body(_, carry):
        t, bv, c = carry
        cand = t + bv
        cnt = cnt_gt_h + packed_count(
            jnp.where(lo_m >= cand.astype(jnp.int16), i16_1, i16_0))
        acc = cnt >= kk
        return jnp.where(acc, cand, t), \
            lax.shift_right_arithmetic(bv, jnp.int32(1)), \
            jnp.where(acc, cnt, c)

    lstar, _, cnt_ge = lax.fori_loop(jnp.int32(0), jnp.int32(16), lo_body,
                                     (t0, bv0, cnt_ge_h))

    thr = lax.shift_left(hstar, jnp.int32(16)) | (
        (lstar ^ jnp.int32(0x8000)) & jnp.int32(0xFFFF))

    maskge = keys >= thr
    wcs = sim * cs
    contrib = jnp.sum(jnp.where(maskge, wcs, 0.0), axis=1)

    ul = ulat_ref[...]
    svd = jnp.sum(il * ul, axis=1)
    base = uq_ref[...] + mq_ref[...] - gm_ref[0, 0]
    pred_ref[...] = jnp.maximum(base + svd + contrib, 0.0)

    @pl.when(step == 0)
    def _():
        reg_ref[...] = jnp.zeros_like(reg_ref)
    reg_ref[...] = reg_ref[...] + (LAMBDA1 * jnp.sum(ul * ul) +
                                   LAMBDA2 * jnp.sum(il * il))

    @pl.when(jnp.max(cnt_ge) > kk)
    def _():
        ni = sim.shape[1]
        gt = keys > thr
        cnt_gt = jnp.sum(gt.astype(jnp.int32), axis=1, keepdims=True,
                         dtype=jnp.int32)
        needed = kk - cnt_gt
        tie = maskge & jnp.logical_not(gt)
        iota = lax.broadcasted_iota(jnp.int32, (1, ni), 1)
        nbits = max(1, (ni - 1).bit_length())

        def idx_body(_, carry):
            p, bv = carry
            cand = p + bv
            h = jnp.sum((tie & (iota < cand)).astype(jnp.int32), axis=1,
                        keepdims=True, dtype=jnp.int32)
            return jnp.where(h < needed, cand, p), lax.shift_right_logical(bv, jnp.int32(1))

        p0 = jnp.zeros((rb, 1), jnp.int32)
        bv0 = jnp.full((rb, 1), jnp.int32(1 << (nbits - 1)))
        pmax, _ = lax.fori_loop(jnp.int32(0), jnp.int32(nbits), idx_body,
                                (p0, bv0))
        sel = gt | (tie & (iota <= pmax))
        contrib2 = jnp.sum(jnp.where(sel, wcs, 0.0), axis=1)
        pred_ref[...] = jnp.maximum(base + svd + contrib2, 0.0)


def kernel(user_indices, item_indices, util_rows, util_cols, util_vals,
           user_emb, item_emb, user_avg, movie_avg, global_mean):
    B = user_indices.shape[0]
    NI, D = item_emb.shape
    D2 = ((D + 127) // 128) * 128
    uep = jnp.pad(user_emb.astype(jnp.float32), ((0, 0), (0, D2 - D)))
    iep = jnp.pad(item_emb.astype(jnp.float32), ((0, 0), (0, D2 - D)))
    ui = user_indices.astype(jnp.int32)
    ii = item_indices.astype(jnp.int32)
    ur = util_rows.astype(jnp.int32)
    uc = util_cols.astype(jnp.int32)
    uv = util_vals.astype(jnp.float32)
    gm = global_mean.astype(jnp.float32)
    gm16 = jnp.broadcast_to(gm.reshape(1), (_L,))

    colp, uq, mq, ulat, ilat = _sc_stage(
        ui, ii, ur, uc, uv, uep, iep, user_avg.astype(jnp.float32),
        movie_avg.astype(jnp.float32), gm16)

    RB = 256
    assert B % RB == 0
    z = lambda i: jnp.int32(0)
    preds, reg = pl.pallas_call(
        functools.partial(_tc_body, k=TOP_K),
        grid=(B // RB,),
        in_specs=[
            pl.BlockSpec((RB, D2), lambda i: (i, z(i))),
            pl.BlockSpec((NI, D2), lambda i: (z(i), z(i))),
            pl.BlockSpec((_NW, NI), lambda i: (z(i), z(i))),
            pl.BlockSpec((RB, D2), lambda i: (i, z(i))),
            pl.BlockSpec((RB,), lambda i: (i,)),
            pl.BlockSpec((RB,), lambda i: (i,)),
            pl.BlockSpec((1, 1), lambda i: (z(i), z(i))),
        ],
        out_specs=[
            pl.BlockSpec((RB,), lambda i: (i,)),
            pl.BlockSpec((1, 1), lambda i: (z(i), z(i))),
        ],
        out_shape=[
            jax.ShapeDtypeStruct((B,), jnp.float32),
            jax.ShapeDtypeStruct((1, 1), jnp.float32),
        ],
    )(ilat, iep, colp, ulat, uq, mq, gm.reshape(1, 1))

    return preds, reg.reshape(())


# speedup vs baseline: 103.8193x; 1.0457x over previous
import functools

import jax
import jax.numpy as jnp
from jax import lax
from jax.experimental import pallas as pl
from jax.experimental.pallas import tpu as pltpu
from jax.experimental.pallas import tpu_sc as plsc

LAMBDA1 = 0.1
LAMBDA2 = 0.1
TOP_K = 256

_NC = 2
_NS = 16
_NW = _NC * _NS
_L = 16

_CHUNK = 10240


def _sc_stage(ui, ii, ur, uc, uv, user_emb, item_emb, user_avg, movie_avg, gm16):
    B = ui.shape[0]
    NNZ = ur.shape[0]
    NU, D = user_emb.shape
    NI = item_emb.shape[0]
    QB = B // _NW
    assert B % (_L * _NW) == 0 and NI % _L == 0 and NU % _L == 0
    assert NNZ % _L == 0

    ch_main = (NNZ // _NW) & ~(_L - 1)
    tail = NNZ - _NW * ch_main
    assert tail % _L == 0 and tail <= _CHUNK
    chunks = [_CHUNK] * (ch_main // _CHUNK)
    if ch_main % _CHUNK:
        chunks.append(ch_main % _CHUNK)

    mesh = plsc.VectorSubcoreMesh(core_axis_name="c", subcore_axis_name="s",
                                  num_cores=_NC, num_subcores=_NS)

    @functools.partial(
        pl.kernel,
        mesh=mesh,
        compiler_params=pltpu.CompilerParams(needs_layout_passes=False),
        out_type=[
            jax.ShapeDtypeStruct((_NW, NI), jnp.float32),
            jax.ShapeDtypeStruct((B,), jnp.float32),
            jax.ShapeDtypeStruct((B,), jnp.float32),
            jax.ShapeDtypeStruct((B, D), jnp.float32),
            jax.ShapeDtypeStruct((B, D), jnp.float32),
        ],
        scratch_types=[
            pltpu.VMEM((NU,), jnp.float32),
            pltpu.VMEM((NI,), jnp.float32),
            pltpu.VMEM((NI,), jnp.float32),
            pltpu.VMEM((_L,), jnp.float32),
            pltpu.VMEM((QB,), jnp.int32),
            pltpu.VMEM((QB,), jnp.int32),
            pltpu.VMEM((QB, D), jnp.float32),
            pltpu.VMEM((QB, D), jnp.float32),
            pltpu.VMEM((QB,), jnp.float32),
            pltpu.VMEM((QB,), jnp.float32),
            pltpu.VMEM((_CHUNK,), jnp.int32),
            pltpu.VMEM((_CHUNK,), jnp.int32),
            pltpu.VMEM((_CHUNK,), jnp.int32),
            pltpu.VMEM((_CHUNK,), jnp.int32),
            pltpu.VMEM((_CHUNK,), jnp.float32),
            pltpu.VMEM((_CHUNK,), jnp.float32),
            pltpu.SemaphoreType.DMA,
            pltpu.SemaphoreType.DMA,
        ],
    )
    def sc_kernel(ui_h, ii_h, ur_h, uc_h, uv_h, ue_h, ie_h, ua_h, ma_h, gm_h,
                  colp_o, uq_o, mq_o, ulat_o, ilat_o,
                  uavg_t, mavg_t, colp_v, gm_v, uidx_v, iidx_v, ulat_v,
                  ilat_v, qa_v, qb_v, rows_a, rows_b, cols_a, cols_b, vals_a,
                  vals_b, sem, nsem):
        wid = lax.axis_index("s") * _NC + lax.axis_index("c")
        qbase = wid * QB
        nnz_off = wid * ch_main
        bufs = ((rows_a, cols_a, vals_a), (rows_b, cols_b, vals_b))

        d_ua = pltpu.async_copy(ua_h, uavg_t, sem)
        d_ma = pltpu.async_copy(ma_h, mavg_t, sem)
        d_gm = pltpu.async_copy(gm_h, gm_v, sem)
        d_ui = pltpu.async_copy(ui_h.at[pl.ds(qbase, QB)], uidx_v, sem)
        d_ii = pltpu.async_copy(ii_h.at[pl.ds(qbase, QB)], iidx_v, sem)
        n0 = chunks[0]
        d_r = pltpu.async_copy(ur_h.at[pl.ds(nnz_off, n0)],
                               rows_a.at[pl.ds(0, n0)], nsem)
        d_c = pltpu.async_copy(uc_h.at[pl.ds(nnz_off, n0)],
                               cols_a.at[pl.ds(0, n0)], nsem)
        d_v = pltpu.async_copy(uv_h.at[pl.ds(nnz_off, n0)],
                               vals_a.at[pl.ds(0, n0)], nsem)
        d_ua.wait()
        d_ma.wait()
        d_gm.wait()
        d_ui.wait()
        d_ii.wait()

        g_u = pltpu.async_copy(ue_h.at[uidx_v], ulat_v, sem)
        g_i = pltpu.async_copy(ie_h.at[iidx_v], ilat_v, sem)
        for j in range(QB // _L):
            sl = pl.ds(j * _L, _L)
            qa_v[sl] = plsc.load_gather(uavg_t, [uidx_v[sl]])
            qb_v[sl] = plsc.load_gather(mavg_t, [iidx_v[sl]])
        w_qa = pltpu.async_copy(qa_v, uq_o.at[pl.ds(qbase, QB)], sem)
        w_qb = pltpu.async_copy(qb_v, mq_o.at[pl.ds(qbase, QB)], sem)
        g_u.wait()
        g_i.wait()
        w_ul = pltpu.async_copy(ulat_v, ulat_o.at[pl.ds(qbase, QB)], sem)
        w_il = pltpu.async_copy(ilat_v, ilat_o.at[pl.ds(qbase, QB)], sem)

        def zero_body(j, carry):
            colp_v[pl.ds(j * _L, _L)] = jnp.zeros((_L,), jnp.float32)
            return carry
        lax.fori_loop(jnp.int32(0), jnp.int32(NI // _L), zero_body,
                      jnp.int32(0))

        gmv = gm_v[...]

        def accum(buf, nvec):
            unroll = 4

            rv, cv, vv = bufs[buf]

            def group(j):
                sl = pl.ds(j * _L, _L)
                r16 = rv[sl]
                c16 = cv[sl]
                v16 = vv[sl]
                ua16 = plsc.load_gather(uavg_t, [r16])
                ma16 = plsc.load_gather(mavg_t, [c16])
                plsc.addupdate_scatter(colp_v, [c16], v16 - ua16 - ma16 - gmv)

            def body(j, carry):
                for u in range(unroll):
                    group(j * unroll + jnp.int32(u))
                return carry

            def body1(j, carry):
                group(j)
                return carry

            lax.fori_loop(jnp.int32(0), jnp.int32(nvec // unroll), body,
                          jnp.int32(0))
            if nvec % unroll:
                lax.fori_loop(jnp.int32(nvec - nvec % unroll),
                              jnp.int32(nvec), body1, jnp.int32(0))

        descs = (d_r, d_c, d_v)
        done = chunks[0]
        for ci, n in enumerate(chunks):
            buf = ci % 2
            for d in descs:
                d.wait()
            if ci + 1 < len(chunks):
                nrv, ncv, nvv = bufs[(ci + 1) % 2]
                nn = chunks[ci + 1]
                noff = nnz_off + done
                done += nn
                descs = (
                    pltpu.async_copy(ur_h.at[pl.ds(noff, nn)],
                                     nrv.at[pl.ds(0, nn)], nsem),
                    pltpu.async_copy(uc_h.at[pl.ds(noff, nn)],
                                     ncv.at[pl.ds(0, nn)], nsem),
                    pltpu.async_copy(uv_h.at[pl.ds(noff, nn)],
                                     nvv.at[pl.ds(0, nn)], nsem),
                )
            accum(buf, n // _L)

        if tail > 0:
            @pl.when(wid == 0)
            def _():
                toff = _NW * ch_main
                pltpu.sync_copy(ur_h.at[pl.ds(toff, tail)],
                                rows_a.at[pl.ds(0, tail)])
                pltpu.sync_copy(uc_h.at[pl.ds(toff, tail)],
                                cols_a.at[pl.ds(0, tail)])
                pltpu.sync_copy(uv_h.at[pl.ds(toff, tail)],
                                vals_a.at[pl.ds(0, tail)])
                accum(0, tail // _L)

        pltpu.sync_copy(colp_v, colp_o.at[wid])
        w_qa.wait()
        w_qb.wait()
        w_ul.wait()
        w_il.wait()

    return sc_kernel(ui, ii, ur, uc, uv, user_emb, item_emb, user_avg,
                     movie_avg, gm16)


def _sortable_keys(x):
    b = lax.bitcast_convert_type(x, jnp.int32)
    return jnp.where(b < 0, b ^ jnp.int32(0x7FFFFFFF), b)


def _tc_body(ilat_ref, iemb_ref, colp_ref, ulat_ref, uq_ref, mq_ref, gm_ref,
             pred_ref, reg_ref, *, k):
    step = pl.program_id(0)
    cs = jnp.sum(colp_ref[...], axis=0, keepdims=True)
    il = ilat_ref[...]
    sim = lax.dot_general(il, iemb_ref[...], (((1,), (1,)), ((), ())),
                          preferred_element_type=jnp.float32)
    keys = _sortable_keys(sim)
    rb = sim.shape[0]
    kk = jnp.int32(k)

    def packed_count(ones):
        w = ones.shape[1]
        while w > 128:
            ones = ones[:, : w // 2] + ones[:, w // 2:]
            w //= 2
        return jnp.sum(ones.astype(jnp.int32), axis=1, keepdims=True,
                       dtype=jnp.int32)

    i16_1 = jnp.int16(1)
    i16_0 = jnp.int16(0)
    hk = lax.shift_right_arithmetic(keys, jnp.int32(16)).astype(jnp.int16)

    def hi_body(_, carry):
        t, bv = carry
        cand = t + bv
        cnt = packed_count(jnp.where(hk >= cand.astype(jnp.int16),
                                     i16_1, i16_0))
        return jnp.where(cnt >= kk, cand, t), \
            lax.shift_right_arithmetic(bv, jnp.int32(1))

    t0 = jnp.full((rb, 1), jnp.int32(-32768))
    bv0 = jnp.full((rb, 1), jnp.int32(32768))
    hstar, _ = lax.fori_loop(jnp.int32(0), jnp.int32(16), hi_body, (t0, bv0))

    lu = ((keys & jnp.int32(0xFFFF)) ^ jnp.int32(0x8000)).astype(jnp.int16)
    h16 = hstar.astype(jnp.int16)
    emask = hk == h16
    cnt_ge_h = packed_count(jnp.where(hk >= h16, i16_1, i16_0))
    cnt_gt_h = cnt_ge_h - packed_count(jnp.where(emask, i16_1, i16_0))
    lo_m = jnp.where(emask, lu, jnp.int16(-32768))

    def lo_body(_, carry):
        t, bv, c = carry
        cand = t + bv
        cnt = cnt_gt_h + packed_count(
            jnp.where(lo_m >= cand.astype(jnp.int16), i16_1, i16_0))
        acc = cnt >= kk
        return jnp.where(acc, cand, t), \
            lax.shift_right_arithmetic(bv, jnp.int32(1)), \
            jnp.where(acc, cnt, c)

    lstar, _, cnt_ge = lax.fori_loop(jnp.int32(0), jnp.int32(16), lo_body,
                                     (t0, bv0, cnt_ge_h))

    thr = lax.shift_left(hstar, jnp.int32(16)) | (
        (lstar ^ jnp.int32(0x8000)) & jnp.int32(0xFFFF))

    maskge = keys >= thr
    wcs = sim * cs
    contrib = jnp.sum(jnp.where(maskge, wcs, 0.0), axis=1)

    ul = ulat_ref[...]
    svd = jnp.sum(il * ul, axis=1)
    base = uq_ref[...] + mq_ref[...] - gm_ref[0, 0]
    pred_ref[...] = jnp.maximum(base + svd + contrib, 0.0)

    @pl.when(step == 0)
    def _():
        reg_ref[...] = jnp.zeros_like(reg_ref)
    reg_ref[...] = reg_ref[...] + (LAMBDA1 * jnp.sum(ul * ul) +
                                   LAMBDA2 * jnp.sum(il * il))

    @pl.when(jnp.max(cnt_ge) > kk)
    def _():
        ni = sim.shape[1]
        gt = keys > thr
        cnt_gt = jnp.sum(gt.astype(jnp.int32), axis=1, keepdims=True,
                         dtype=jnp.int32)
        needed = kk - cnt_gt
        tie = maskge & jnp.logical_not(gt)
        iota = lax.broadcasted_iota(jnp.int32, (1, ni), 1)
        nbits = max(1, (ni - 1).bit_length())

        def idx_body(_, carry):
            p, bv = carry
            cand = p + bv
            h = jnp.sum((tie & (iota < cand)).astype(jnp.int32), axis=1,
                        keepdims=True, dtype=jnp.int32)
            return jnp.where(h < needed, cand, p), lax.shift_right_logical(bv, jnp.int32(1))

        p0 = jnp.zeros((rb, 1), jnp.int32)
        bv0 = jnp.full((rb, 1), jnp.int32(1 << (nbits - 1)))
        pmax, _ = lax.fori_loop(jnp.int32(0), jnp.int32(nbits), idx_body,
                                (p0, bv0))
        sel = gt | (tie & (iota <= pmax))
        contrib2 = jnp.sum(jnp.where(sel, wcs, 0.0), axis=1)
        pred_ref[...] = jnp.maximum(base + svd + contrib2, 0.0)


def kernel(user_indices, item_indices, util_rows, util_cols, util_vals,
           user_emb, item_emb, user_avg, movie_avg, global_mean):
    B = user_indices.shape[0]
    NI, D = item_emb.shape
    D2 = ((D + 127) // 128) * 128
    uep = jnp.pad(user_emb.astype(jnp.float32), ((0, 0), (0, D2 - D)))
    iep = jnp.pad(item_emb.astype(jnp.float32), ((0, 0), (0, D2 - D)))
    ui = user_indices.astype(jnp.int32)
    ii = item_indices.astype(jnp.int32)
    ur = util_rows.astype(jnp.int32)
    uc = util_cols.astype(jnp.int32)
    uv = util_vals.astype(jnp.float32)
    gm = global_mean.astype(jnp.float32)
    gm16 = jnp.broadcast_to(gm.reshape(1), (_L,))

    colp, uq, mq, ulat, ilat = _sc_stage(
        ui, ii, ur, uc, uv, uep, iep, user_avg.astype(jnp.float32),
        movie_avg.astype(jnp.float32), gm16)

    RB = 256
    assert B % RB == 0
    z = lambda i: jnp.int32(0)
    preds, reg = pl.pallas_call(
        functools.partial(_tc_body, k=TOP_K),
        grid=(B // RB,),
        in_specs=[
            pl.BlockSpec((RB, D2), lambda i: (i, z(i))),
            pl.BlockSpec((NI, D2), lambda i: (z(i), z(i))),
            pl.BlockSpec((_NW, NI), lambda i: (z(i), z(i))),
            pl.BlockSpec((RB, D2), lambda i: (i, z(i))),
            pl.BlockSpec((RB,), lambda i: (i,)),
            pl.BlockSpec((RB,), lambda i: (i,)),
            pl.BlockSpec((1, 1), lambda i: (z(i), z(i))),
        ],
        out_specs=[
            pl.BlockSpec((RB,), lambda i: (i,)),
            pl.BlockSpec((1, 1), lambda i: (z(i), z(i))),
        ],
        out_shape=[
            jax.ShapeDtypeStruct((B,), jnp.float32),
            jax.ShapeDtypeStruct((1, 1), jnp.float32),
        ],
    )(ilat, iep, colp, ulat, uq, mq, gm.reshape(1, 1))

    return preds, reg.reshape(())


# speedup vs baseline: 111.5056x; 1.0740x over previous
import functools

import jax
import jax.numpy as jnp
from jax import lax
from jax.experimental import pallas as pl
from jax.experimental.pallas import tpu as pltpu
from jax.experimental.pallas import tpu_sc as plsc

LAMBDA1 = 0.1
LAMBDA2 = 0.1
TOP_K = 256

_NC = 2
_NS = 16
_NW = _NC * _NS
_L = 16

_CHUNK = 10240


def _sc_stage(ui, ii, ur, uc, uv, user_emb, item_emb, user_avg, movie_avg, gm16):
    B = ui.shape[0]
    NNZ = ur.shape[0]
    NU, D = user_emb.shape
    NI = item_emb.shape[0]
    QB = B // _NW
    assert B % (_L * _NW) == 0 and NI % _L == 0 and NU % _L == 0
    assert NNZ % _L == 0

    ch_main = (NNZ // _NW) & ~(_L - 1)
    tail = NNZ - _NW * ch_main
    assert tail % _L == 0 and tail <= _CHUNK
    chunks = [_CHUNK] * (ch_main // _CHUNK)
    if ch_main % _CHUNK:
        chunks.append(ch_main % _CHUNK)

    mesh = plsc.VectorSubcoreMesh(core_axis_name="c", subcore_axis_name="s",
                                  num_cores=_NC, num_subcores=_NS)

    @functools.partial(
        pl.kernel,
        mesh=mesh,
        compiler_params=pltpu.CompilerParams(needs_layout_passes=False),
        out_type=[
            jax.ShapeDtypeStruct((_NW, NI), jnp.float32),
            jax.ShapeDtypeStruct((B,), jnp.float32),
            jax.ShapeDtypeStruct((B,), jnp.float32),
            jax.ShapeDtypeStruct((B, D), jnp.float32),
            jax.ShapeDtypeStruct((B, D), jnp.float32),
        ],
        scratch_types=[
            pltpu.VMEM((NU,), jnp.float32),
            pltpu.VMEM((NI,), jnp.float32),
            pltpu.VMEM((NI,), jnp.float32),
            pltpu.VMEM((_L,), jnp.float32),
            pltpu.VMEM((QB,), jnp.int32),
            pltpu.VMEM((QB,), jnp.int32),
            pltpu.VMEM((QB, D), jnp.float32),
            pltpu.VMEM((QB, D), jnp.float32),
            pltpu.VMEM((QB,), jnp.float32),
            pltpu.VMEM((QB,), jnp.float32),
            pltpu.VMEM((_CHUNK,), jnp.int32),
            pltpu.VMEM((_CHUNK,), jnp.int32),
            pltpu.VMEM((_CHUNK,), jnp.int32),
            pltpu.VMEM((_CHUNK,), jnp.int32),
            pltpu.VMEM((_CHUNK,), jnp.float32),
            pltpu.VMEM((_CHUNK,), jnp.float32),
            pltpu.SemaphoreType.DMA,
            pltpu.SemaphoreType.DMA,
        ],
    )
    def sc_kernel(ui_h, ii_h, ur_h, uc_h, uv_h, ue_h, ie_h, ua_h, ma_h, gm_h,
                  colp_o, uq_o, mq_o, ulat_o, ilat_o,
                  uavg_t, mavg_t, colp_v, gm_v, uidx_v, iidx_v, ulat_v,
                  ilat_v, qa_v, qb_v, rows_a, rows_b, cols_a, cols_b, vals_a,
                  vals_b, sem, nsem):
        wid = lax.axis_index("s") * _NC + lax.axis_index("c")
        qbase = wid * QB
        nnz_off = wid * ch_main
        bufs = ((rows_a, cols_a, vals_a), (rows_b, cols_b, vals_b))

        d_ua = pltpu.async_copy(ua_h, uavg_t, sem)
        d_ma = pltpu.async_copy(ma_h, mavg_t, sem)
        d_gm = pltpu.async_copy(gm_h, gm_v, sem)
        d_ui = pltpu.async_copy(ui_h.at[pl.ds(qbase, QB)], uidx_v, sem)
        d_ii = pltpu.async_copy(ii_h.at[pl.ds(qbase, QB)], iidx_v, sem)
        n0 = chunks[0]
        d_r = pltpu.async_copy(ur_h.at[pl.ds(nnz_off, n0)],
                               rows_a.at[pl.ds(0, n0)], nsem)
        d_c = pltpu.async_copy(uc_h.at[pl.ds(nnz_off, n0)],
                               cols_a.at[pl.ds(0, n0)], nsem)
        d_v = pltpu.async_copy(uv_h.at[pl.ds(nnz_off, n0)],
                               vals_a.at[pl.ds(0, n0)], nsem)
        d_ua.wait()
        d_ma.wait()
        d_gm.wait()
        d_ui.wait()
        d_ii.wait()

        g_u = pltpu.async_copy(ue_h.at[uidx_v], ulat_v, sem)
        g_i = pltpu.async_copy(ie_h.at[iidx_v], ilat_v, sem)
        for j in range(QB // _L):
            sl = pl.ds(j * _L, _L)
            qa_v[sl] = plsc.load_gather(uavg_t, [uidx_v[sl]])
            qb_v[sl] = plsc.load_gather(mavg_t, [iidx_v[sl]])
        w_qa = pltpu.async_copy(qa_v, uq_o.at[pl.ds(qbase, QB)], sem)
        w_qb = pltpu.async_copy(qb_v, mq_o.at[pl.ds(qbase, QB)], sem)
        g_u.wait()
        g_i.wait()
        w_ul = pltpu.async_copy(ulat_v, ulat_o.at[pl.ds(qbase, QB)], sem)
        w_il = pltpu.async_copy(ilat_v, ilat_o.at[pl.ds(qbase, QB)], sem)

        def zero_body(j, carry):
            colp_v[pl.ds(j * _L, _L)] = jnp.zeros((_L,), jnp.float32)
            return carry
        lax.fori_loop(jnp.int32(0), jnp.int32(NI // _L), zero_body,
                      jnp.int32(0))

        gmv = gm_v[...]

        def accum(buf, nvec):
            unroll = 4

            rv, cv, vv = bufs[buf]

            def group(j):
                sl = pl.ds(j * _L, _L)
                r16 = rv[sl]
                c16 = cv[sl]
                v16 = vv[sl]
                ua16 = plsc.load_gather(uavg_t, [r16])
                ma16 = plsc.load_gather(mavg_t, [c16])
                plsc.addupdate_scatter(colp_v, [c16], v16 - ua16 - ma16 - gmv)

            def body(j, carry):
                for u in range(unroll):
                    group(j * unroll + jnp.int32(u))
                return carry

            def body1(j, carry):
                group(j)
                return carry

            lax.fori_loop(jnp.int32(0), jnp.int32(nvec // unroll), body,
                          jnp.int32(0))
            if nvec % unroll:
                lax.fori_loop(jnp.int32(nvec - nvec % unroll),
                              jnp.int32(nvec), body1, jnp.int32(0))

        descs = (d_r, d_c, d_v)
        done = chunks[0]
        for ci, n in enumerate(chunks):
            buf = ci % 2
            for d in descs:
                d.wait()
            if ci + 1 < len(chunks):
                nrv, ncv, nvv = bufs[(ci + 1) % 2]
                nn = chunks[ci + 1]
                noff = nnz_off + done
                done += nn
                descs = (
                    pltpu.async_copy(ur_h.at[pl.ds(noff, nn)],
                                     nrv.at[pl.ds(0, nn)], nsem),
                    pltpu.async_copy(uc_h.at[pl.ds(noff, nn)],
                                     ncv.at[pl.ds(0, nn)], nsem),
                    pltpu.async_copy(uv_h.at[pl.ds(noff, nn)],
                                     nvv.at[pl.ds(0, nn)], nsem),
                )
            accum(buf, n // _L)

        if tail > 0:
            @pl.when(wid == 0)
            def _():
                toff = _NW * ch_main
                pltpu.sync_copy(ur_h.at[pl.ds(toff, tail)],
                                rows_a.at[pl.ds(0, tail)])
                pltpu.sync_copy(uc_h.at[pl.ds(toff, tail)],
                                cols_a.at[pl.ds(0, tail)])
                pltpu.sync_copy(uv_h.at[pl.ds(toff, tail)],
                                vals_a.at[pl.ds(0, tail)])
                accum(0, tail // _L)

        pltpu.sync_copy(colp_v, colp_o.at[wid])
        w_qa.wait()
        w_qb.wait()
        w_ul.wait()
        w_il.wait()

    return sc_kernel(ui, ii, ur, uc, uv, user_emb, item_emb, user_avg,
                     movie_avg, gm16)


def _sortable_keys(x):
    b = lax.bitcast_convert_type(x, jnp.int32)
    return jnp.where(b < 0, b ^ jnp.int32(0x7FFFFFFF), b)


def _tc_body(ilat_ref, iemb_ref, colp_ref, ulat_ref, uq_ref, mq_ref, gm_ref,
             pred_ref, reg_ref, *, k):
    step = pl.program_id(0)
    cs = jnp.sum(colp_ref[...], axis=0, keepdims=True)
    il = ilat_ref[...]
    sim = lax.dot_general(il, iemb_ref[...], (((1,), (1,)), ((), ())),
                          preferred_element_type=jnp.float32)
    keys = _sortable_keys(sim)
    rb = sim.shape[0]
    kk = jnp.int32(k)

    def packed_count(ones):
        w = ones.shape[1]
        while w > 128:
            ones = ones[:, : w // 2] + ones[:, w // 2:]
            w //= 2
        return jnp.sum(ones.astype(jnp.int32), axis=1, keepdims=True,
                       dtype=jnp.int32)

    i16_1 = jnp.int16(1)
    i16_0 = jnp.int16(0)
    hk = lax.shift_right_arithmetic(keys, jnp.int32(16)).astype(jnp.int16)

    def hi_body(_, carry):
        t, bv = carry
        cand = t + bv
        cnt = packed_count(jnp.where(hk >= cand.astype(jnp.int16),
                                     i16_1, i16_0))
        return jnp.where(cnt >= kk, cand, t), \
            lax.shift_right_arithmetic(bv, jnp.int32(1))

    t0 = jnp.full((rb, 1), jnp.int32(-32768))
    bv0 = jnp.full((rb, 1), jnp.int32(32768))
    hstar, _ = lax.fori_loop(jnp.int32(0), jnp.int32(16), hi_body, (t0, bv0))

    lu = ((keys & jnp.int32(0xFFFF)) ^ jnp.int32(0x8000)).astype(jnp.int16)
    h16 = hstar.astype(jnp.int16)
    emask = hk == h16
    cnt_ge_h = packed_count(jnp.where(hk >= h16, i16_1, i16_0))
    cnt_gt_h = cnt_ge_h - packed_count(jnp.where(emask, i16_1, i16_0))
    lo_m = jnp.where(emask, lu, jnp.int16(-32768))

    def lo_body(_, carry):
        t, bv, c = carry
        cand = t + bv
        cnt = cnt_gt_h + packed_count(
            jnp.where(lo_m >= cand.astype(jnp.int16), i16_1, i16_0))
        acc = cnt >= kk
        return jnp.where(acc, cand, t), \
            lax.shift_right_arithmetic(bv, jnp.int32(1)), \
            jnp.where(acc, cnt, c)

    lstar, _, cnt_ge = lax.fori_loop(jnp.int32(0), jnp.int32(16), lo_body,
                                     (t0, bv0, cnt_ge_h))

    thr = lax.shift_left(hstar, jnp.int32(16)) | (
        (lstar ^ jnp.int32(0x8000)) & jnp.int32(0xFFFF))

    maskge = keys >= thr
    wcs = sim * cs
    contrib = jnp.sum(jnp.where(maskge, wcs, 0.0), axis=1)

    ul = ulat_ref[...]
    svd = jnp.sum(il * ul, axis=1)
    base = uq_ref[...] + mq_ref[...] - gm_ref[0, 0]
    pred_ref[...] = jnp.maximum(base + svd + contrib, 0.0)

    @pl.when(step == 0)
    def _():
        reg_ref[...] = jnp.zeros_like(reg_ref)
    reg_ref[...] = reg_ref[...] + (LAMBDA1 * jnp.sum(ul * ul) +
                                   LAMBDA2 * jnp.sum(il * il))

    @pl.when(jnp.max(cnt_ge) > kk)
    def _():
        ni = sim.shape[1]
        gt = keys > thr
        cnt_gt = jnp.sum(gt.astype(jnp.int32), axis=1, keepdims=True,
                         dtype=jnp.int32)
        needed = kk - cnt_gt
        tie = maskge & jnp.logical_not(gt)
        iota = lax.broadcasted_iota(jnp.int32, (1, ni), 1)
        nbits = max(1, (ni - 1).bit_length())

        def idx_body(_, carry):
            p, bv = carry
            cand = p + bv
            h = jnp.sum((tie & (iota < cand)).astype(jnp.int32), axis=1,
                        keepdims=True, dtype=jnp.int32)
            return jnp.where(h < needed, cand, p), lax.shift_right_logical(bv, jnp.int32(1))

        p0 = jnp.zeros((rb, 1), jnp.int32)
        bv0 = jnp.full((rb, 1), jnp.int32(1 << (nbits - 1)))
        pmax, _ = lax.fori_loop(jnp.int32(0), jnp.int32(nbits), idx_body,
                                (p0, bv0))
        sel = gt | (tie & (iota <= pmax))
        contrib2 = jnp.sum(jnp.where(sel, wcs, 0.0), axis=1)
        pred_ref[...] = jnp.maximum(base + svd + contrib2, 0.0)


def kernel(user_indices, item_indices, util_rows, util_cols, util_vals,
           user_emb, item_emb, user_avg, movie_avg, global_mean):
    B = user_indices.shape[0]
    NI, D = item_emb.shape
    D2 = ((D + 127) // 128) * 128
    uep = jnp.pad(user_emb.astype(jnp.float32), ((0, 0), (0, D2 - D)))
    iep = jnp.pad(item_emb.astype(jnp.float32), ((0, 0), (0, D2 - D)))
    ui = user_indices.astype(jnp.int32)
    ii = item_indices.astype(jnp.int32)
    ur = util_rows.astype(jnp.int32)
    uc = util_cols.astype(jnp.int32)
    uv = util_vals.astype(jnp.float32)
    gm = global_mean.astype(jnp.float32)
    gm16 = jnp.broadcast_to(gm.reshape(1), (_L,))

    colp, uq, mq, ulat, ilat = _sc_stage(
        ui, ii, ur, uc, uv, uep, iep, user_avg.astype(jnp.float32),
        movie_avg.astype(jnp.float32), gm16)

    RB = 512
    assert B % RB == 0
    z = lambda i: jnp.int32(0)
    preds, reg = pl.pallas_call(
        functools.partial(_tc_body, k=TOP_K),
        grid=(B // RB,),
        in_specs=[
            pl.BlockSpec((RB, D2), lambda i: (i, z(i))),
            pl.BlockSpec((NI, D2), lambda i: (z(i), z(i))),
            pl.BlockSpec((_NW, NI), lambda i: (z(i), z(i))),
            pl.BlockSpec((RB, D2), lambda i: (i, z(i))),
            pl.BlockSpec((RB,), lambda i: (i,)),
            pl.BlockSpec((RB,), lambda i: (i,)),
            pl.BlockSpec((1, 1), lambda i: (z(i), z(i))),
        ],
        out_specs=[
            pl.BlockSpec((RB,), lambda i: (i,)),
            pl.BlockSpec((1, 1), lambda i: (z(i), z(i))),
        ],
        out_shape=[
            jax.ShapeDtypeStruct((B,), jnp.float32),
            jax.ShapeDtypeStruct((1, 1), jnp.float32),
        ],
    )(ilat, iep, colp, ulat, uq, mq, gm.reshape(1, 1))

    return preds, reg.reshape(())


# speedup vs baseline: 111.8412x; 1.0030x over previous
import functools

import jax
import jax.numpy as jnp
from jax import lax
from jax.experimental import pallas as pl
from jax.experimental.pallas import tpu as pltpu
from jax.experimental.pallas import tpu_sc as plsc

LAMBDA1 = 0.1
LAMBDA2 = 0.1
TOP_K = 256

_NC = 2
_NS = 16
_NW = _NC * _NS
_L = 16

_CHUNK = 10240


def _sc_stage(ui, ii, ur, uc, uv, user_emb, item_emb, user_avg, movie_avg, gm16):
    B = ui.shape[0]
    NNZ = ur.shape[0]
    NU, D = user_emb.shape
    NI = item_emb.shape[0]
    QB = B // _NW
    assert B % (_L * _NW) == 0 and NI % _L == 0 and NU % _L == 0
    assert NNZ % _L == 0

    ch_main = (NNZ // _NW) & ~(_L - 1)
    tail = NNZ - _NW * ch_main
    assert tail % _L == 0 and tail <= _CHUNK
    chunks = [_CHUNK] * (ch_main // _CHUNK)
    if ch_main % _CHUNK:
        chunks.append(ch_main % _CHUNK)

    mesh = plsc.VectorSubcoreMesh(core_axis_name="c", subcore_axis_name="s",
                                  num_cores=_NC, num_subcores=_NS)

    @functools.partial(
        pl.kernel,
        mesh=mesh,
        compiler_params=pltpu.CompilerParams(needs_layout_passes=False),
        out_type=[
            jax.ShapeDtypeStruct((_NW, NI), jnp.float32),
            jax.ShapeDtypeStruct((B,), jnp.float32),
            jax.ShapeDtypeStruct((B,), jnp.float32),
            jax.ShapeDtypeStruct((B, D), jnp.float32),
            jax.ShapeDtypeStruct((B, D), jnp.float32),
        ],
        scratch_types=[
            pltpu.VMEM((NU,), jnp.float32),
            pltpu.VMEM((NI,), jnp.float32),
            pltpu.VMEM((NI,), jnp.float32),
            pltpu.VMEM((_L,), jnp.float32),
            pltpu.VMEM((QB,), jnp.int32),
            pltpu.VMEM((QB,), jnp.int32),
            pltpu.VMEM((QB, D), jnp.float32),
            pltpu.VMEM((QB, D), jnp.float32),
            pltpu.VMEM((QB,), jnp.float32),
            pltpu.VMEM((QB,), jnp.float32),
            pltpu.VMEM((_CHUNK,), jnp.int32),
            pltpu.VMEM((_CHUNK,), jnp.int32),
            pltpu.VMEM((_CHUNK,), jnp.int32),
            pltpu.VMEM((_CHUNK,), jnp.int32),
            pltpu.VMEM((_CHUNK,), jnp.float32),
            pltpu.VMEM((_CHUNK,), jnp.float32),
            pltpu.SemaphoreType.DMA,
            pltpu.SemaphoreType.DMA,
        ],
    )
    def sc_kernel(ui_h, ii_h, ur_h, uc_h, uv_h, ue_h, ie_h, ua_h, ma_h, gm_h,
                  colp_o, uq_o, mq_o, ulat_o, ilat_o,
                  uavg_t, mavg_t, colp_v, gm_v, uidx_v, iidx_v, ulat_v,
                  ilat_v, qa_v, qb_v, rows_a, rows_b, cols_a, cols_b, vals_a,
                  vals_b, sem, nsem):
        wid = lax.axis_index("s") * _NC + lax.axis_index("c")
        qbase = wid * QB
        nnz_off = wid * ch_main
        bufs = ((rows_a, cols_a, vals_a), (rows_b, cols_b, vals_b))

        d_ua = pltpu.async_copy(ua_h, uavg_t, sem)
        d_ma = pltpu.async_copy(ma_h, mavg_t, sem)
        d_gm = pltpu.async_copy(gm_h, gm_v, sem)
        d_ui = pltpu.async_copy(ui_h.at[pl.ds(qbase, QB)], uidx_v, sem)
        d_ii = pltpu.async_copy(ii_h.at[pl.ds(qbase, QB)], iidx_v, sem)
        n0 = chunks[0]
        d_r = pltpu.async_copy(ur_h.at[pl.ds(nnz_off, n0)],
                               rows_a.at[pl.ds(0, n0)], nsem)
        d_c = pltpu.async_copy(uc_h.at[pl.ds(nnz_off, n0)],
                               cols_a.at[pl.ds(0, n0)], nsem)
        d_v = pltpu.async_copy(uv_h.at[pl.ds(nnz_off, n0)],
                               vals_a.at[pl.ds(0, n0)], nsem)
        d_ua.wait()
        d_ma.wait()
        d_gm.wait()
        d_ui.wait()
        d_ii.wait()

        g_u = pltpu.async_copy(ue_h.at[uidx_v], ulat_v, sem)
        g_i = pltpu.async_copy(ie_h.at[iidx_v], ilat_v, sem)
        for j in range(QB // _L):
            sl = pl.ds(j * _L, _L)
            qa_v[sl] = plsc.load_gather(uavg_t, [uidx_v[sl]])
            qb_v[sl] = plsc.load_gather(mavg_t, [iidx_v[sl]])
        w_qa = pltpu.async_copy(qa_v, uq_o.at[pl.ds(qbase, QB)], sem)
        w_qb = pltpu.async_copy(qb_v, mq_o.at[pl.ds(qbase, QB)], sem)
        g_u.wait()
        g_i.wait()
        w_ul = pltpu.async_copy(ulat_v, ulat_o.at[pl.ds(qbase, QB)], sem)
        w_il = pltpu.async_copy(ilat_v, ilat_o.at[pl.ds(qbase, QB)], sem)

        def zero_body(j, carry):
            colp_v[pl.ds(j * _L, _L)] = jnp.zeros((_L,), jnp.float32)
            return carry
        lax.fori_loop(jnp.int32(0), jnp.int32(NI // _L), zero_body,
                      jnp.int32(0))

        gmv = gm_v[...]

        def accum(buf, nvec):
            unroll = 4

            rv, cv, vv = bufs[buf]

            def group(j):
                sl = pl.ds(j * _L, _L)
                r16 = rv[sl]
                c16 = cv[sl]
                v16 = vv[sl]
                ua16 = plsc.load_gather(uavg_t, [r16])
                ma16 = plsc.load_gather(mavg_t, [c16])
                plsc.addupdate_scatter(colp_v, [c16], v16 - ua16 - ma16 - gmv)

            def body(j, carry):
                for u in range(unroll):
                    group(j * unroll + jnp.int32(u))
                return carry

            def body1(j, carry):
                group(j)
                return carry

            lax.fori_loop(jnp.int32(0), jnp.int32(nvec // unroll), body,
                          jnp.int32(0))
            if nvec % unroll:
                lax.fori_loop(jnp.int32(nvec - nvec % unroll),
                              jnp.int32(nvec), body1, jnp.int32(0))

        descs = (d_r, d_c, d_v)
        done = chunks[0]
        for ci, n in enumerate(chunks):
            buf = ci % 2
            for d in descs:
                d.wait()
            if ci + 1 < len(chunks):
                nrv, ncv, nvv = bufs[(ci + 1) % 2]
                nn = chunks[ci + 1]
                noff = nnz_off + done
                done += nn
                descs = (
                    pltpu.async_copy(ur_h.at[pl.ds(noff, nn)],
                                     nrv.at[pl.ds(0, nn)], nsem),
                    pltpu.async_copy(uc_h.at[pl.ds(noff, nn)],
                                     ncv.at[pl.ds(0, nn)], nsem),
                    pltpu.async_copy(uv_h.at[pl.ds(noff, nn)],
                                     nvv.at[pl.ds(0, nn)], nsem),
                )
            accum(buf, n // _L)

        if tail > 0:
            @pl.when(wid == 0)
            def _():
                toff = _NW * ch_main
                pltpu.sync_copy(ur_h.at[pl.ds(toff, tail)],
                                rows_a.at[pl.ds(0, tail)])
                pltpu.sync_copy(uc_h.at[pl.ds(toff, tail)],
                                cols_a.at[pl.ds(0, tail)])
                pltpu.sync_copy(uv_h.at[pl.ds(toff, tail)],
                                vals_a.at[pl.ds(0, tail)])
                accum(0, tail // _L)

        pltpu.sync_copy(colp_v, colp_o.at[wid])
        w_qa.wait()
        w_qb.wait()
        w_ul.wait()
        w_il.wait()

    return sc_kernel(ui, ii, ur, uc, uv, user_emb, item_emb, user_avg,
                     movie_avg, gm16)


def _sortable_keys(x):
    b = lax.bitcast_convert_type(x, jnp.int32)
    return jnp.where(b < 0, b ^ jnp.int32(0x7FFFFFFF), b)


def _tc_body(ilat_ref, iemb_ref, colp_ref, ulat_ref, uq_ref, mq_ref, gm_ref,
             pred_ref, reg_ref, *, k):
    step = pl.program_id(0)
    cs = jnp.sum(colp_ref[...], axis=0, keepdims=True)
    il = ilat_ref[...]
    sim = lax.dot_general(il, iemb_ref[...], (((1,), (1,)), ((), ())),
                          preferred_element_type=jnp.float32)
    keys = _sortable_keys(sim)
    rb = sim.shape[0]
    kk = jnp.int32(k)

    def packed_count(ones):
        w = ones.shape[1]
        while w > 128:
            ones = ones[:, : w // 2] + ones[:, w // 2:]
            w //= 2
        return jnp.sum(ones.astype(jnp.int32), axis=1, keepdims=True,
                       dtype=jnp.int32)

    i16_1 = jnp.int16(1)
    i16_0 = jnp.int16(0)
    hk = lax.shift_right_arithmetic(keys, jnp.int32(16)).astype(jnp.int16)

    def hi_body(_, carry):
        t, bv, c = carry
        cand = t + bv
        cnt = packed_count(jnp.where(hk >= cand.astype(jnp.int16),
                                     i16_1, i16_0))
        acc = cnt >= kk
        return jnp.where(acc, cand, t), \
            lax.shift_right_arithmetic(bv, jnp.int32(1)), \
            jnp.where(acc, cnt, c)

    t0 = jnp.full((rb, 1), jnp.int32(-32768))
    bv0 = jnp.full((rb, 1), jnp.int32(32768))
    nrow = jnp.full((rb, 1), jnp.int32(sim.shape[1]))
    hstar, _, cnt_ge_h = lax.fori_loop(jnp.int32(0), jnp.int32(16), hi_body,
                                       (t0, bv0, nrow))

    lu = keys.astype(jnp.int16) ^ jnp.int16(-32768)
    h16 = hstar.astype(jnp.int16)
    emask = hk == h16
    cnt_gt_h = cnt_ge_h - packed_count(jnp.where(emask, i16_1, i16_0))
    lo_m = jnp.where(emask, lu, jnp.int16(-32768))

    def lo_body(_, carry):
        t, bv, c = carry
        cand = t + bv
        cnt = cnt_gt_h + packed_count(
            jnp.where(lo_m >= cand.astype(jnp.int16), i16_1, i16_0))
        acc = cnt >= kk
        return jnp.where(acc, cand, t), \
            lax.shift_right_arithmetic(bv, jnp.int32(1)), \
            jnp.where(acc, cnt, c)

    lstar, _, cnt_ge = lax.fori_loop(jnp.int32(0), jnp.int32(16), lo_body,
                                     (t0, bv0, cnt_ge_h))

    thr = lax.shift_left(hstar, jnp.int32(16)) | (
        (lstar ^ jnp.int32(0x8000)) & jnp.int32(0xFFFF))

    maskge = keys >= thr
    wcs = sim * cs
    contrib = jnp.sum(jnp.where(maskge, wcs, 0.0), axis=1)

    ul = ulat_ref[...]
    svd = jnp.sum(il * ul, axis=1)
    base = uq_ref[...] + mq_ref[...] - gm_ref[0, 0]
    pred_ref[...] = jnp.maximum(base + svd + contrib, 0.0)

    @pl.when(step == 0)
    def _():
        reg_ref[...] = jnp.zeros_like(reg_ref)
    reg_ref[...] = reg_ref[...] + (LAMBDA1 * jnp.sum(ul * ul) +
                                   LAMBDA2 * jnp.sum(il * il))

    @pl.when(jnp.max(cnt_ge) > kk)
    def _():
        ni = sim.shape[1]
        gt = keys > thr
        cnt_gt = jnp.sum(gt.astype(jnp.int32), axis=1, keepdims=True,
                         dtype=jnp.int32)
        needed = kk - cnt_gt
        tie = maskge & jnp.logical_not(gt)
        iota = lax.broadcasted_iota(jnp.int32, (1, ni), 1)
        nbits = max(1, (ni - 1).bit_length())

        def idx_body(_, carry):
            p, bv = carry
            cand = p + bv
            h = jnp.sum((tie & (iota < cand)).astype(jnp.int32), axis=1,
                        keepdims=True, dtype=jnp.int32)
            return jnp.where(h < needed, cand, p), lax.shift_right_logical(bv, jnp.int32(1))

        p0 = jnp.zeros((rb, 1), jnp.int32)
        bv0 = jnp.full((rb, 1), jnp.int32(1 << (nbits - 1)))
        pmax, _ = lax.fori_loop(jnp.int32(0), jnp.int32(nbits), idx_body,
                                (p0, bv0))
        sel = gt | (tie & (iota <= pmax))
        contrib2 = jnp.sum(jnp.where(sel, wcs, 0.0), axis=1)
        pred_ref[...] = jnp.maximum(base + svd + contrib2, 0.0)


def kernel(user_indices, item_indices, util_rows, util_cols, util_vals,
           user_emb, item_emb, user_avg, movie_avg, global_mean):
    B = user_indices.shape[0]
    NI, D = item_emb.shape
    D2 = ((D + 127) // 128) * 128
    uep = jnp.pad(user_emb.astype(jnp.float32), ((0, 0), (0, D2 - D)))
    iep = jnp.pad(item_emb.astype(jnp.float32), ((0, 0), (0, D2 - D)))
    ui = user_indices.astype(jnp.int32)
    ii = item_indices.astype(jnp.int32)
    ur = util_rows.astype(jnp.int32)
    uc = util_cols.astype(jnp.int32)
    uv = util_vals.astype(jnp.float32)
    gm = global_mean.astype(jnp.float32)
    gm16 = jnp.broadcast_to(gm.reshape(1), (_L,))

    colp, uq, mq, ulat, ilat = _sc_stage(
        ui, ii, ur, uc, uv, uep, iep, user_avg.astype(jnp.float32),
        movie_avg.astype(jnp.float32), gm16)

    RB = 512
    assert B % RB == 0
    z = lambda i: jnp.int32(0)
    preds, reg = pl.pallas_call(
        functools.partial(_tc_body, k=TOP_K),
        grid=(B // RB,),
        in_specs=[
            pl.BlockSpec((RB, D2), lambda i: (i, z(i))),
            pl.BlockSpec((NI, D2), lambda i: (z(i), z(i))),
            pl.BlockSpec((_NW, NI), lambda i: (z(i), z(i))),
            pl.BlockSpec((RB, D2), lambda i: (i, z(i))),
            pl.BlockSpec((RB,), lambda i: (i,)),
            pl.BlockSpec((RB,), lambda i: (i,)),
            pl.BlockSpec((1, 1), lambda i: (z(i), z(i))),
        ],
        out_specs=[
            pl.BlockSpec((RB,), lambda i: (i,)),
            pl.BlockSpec((1, 1), lambda i: (z(i), z(i))),
        ],
        out_shape=[
            jax.ShapeDtypeStruct((B,), jnp.float32),
            jax.ShapeDtypeStruct((1, 1), jnp.float32),
        ],
    )(ilat, iep, colp, ulat, uq, mq, gm.reshape(1, 1))

    return preds, reg.reshape(())
